# Initial kernel scaffold; baseline (speedup 1.0000x reference)
import jax
import jax.numpy as jnp
from jax.experimental import pallas as pl


def kernel(prediction, target, mask):
    raise NotImplementedError("write your pallas kernel here")


# speedup vs baseline: 1.5499x; 1.5499x over previous
import functools

import jax
import jax.numpy as jnp
from jax import lax
from jax.experimental import pallas as pl
from jax.experimental.pallas import tpu as pltpu
from jax.experimental.pallas import tpu_sc as plsc

_L = 16
_ROWS = 16
_COLS = 4096
_HALF = _COLS // 2
_HALF_CHUNKS = _HALF // _L

_mesh = plsc.VectorSubcoreMesh(core_axis_name="c", subcore_axis_name="s")


def _popcount(mb):
    p = plsc.all_reduce_population_count(mb)
    return lax.squeeze(lax.slice_in_dim(p, 0, 1), (0,))


@functools.partial(
    pl.kernel,
    out_type=jax.ShapeDtypeStruct((32, 2, _L), jnp.float32),
    mesh=_mesh,
    scratch_types=[
        pltpu.VMEM((_HALF,), jnp.float32),
        pltpu.VMEM((_COLS + _L,), jnp.float32),
        pltpu.VMEM((_COLS,), jnp.int32),
        pltpu.VMEM((2, _L), jnp.float32),
    ],
    compiler_params=pltpu.CompilerParams(needs_layout_passes=False),
)
def _sc_partials(pred_hbm, tgt_hbm, mask_hbm, out_hbm, pred_v, tgt_v, mask_v, out_v):
    c = lax.axis_index("c")
    s = lax.axis_index("s")

    pltpu.sync_copy(pred_hbm.at[s, pl.ds(c * _HALF, _HALF)], pred_v)
    pltpu.sync_copy(tgt_hbm.at[s], tgt_v.at[pl.ds(0, _COLS)])
    pltpu.sync_copy(mask_hbm.at[s], mask_v)

    def pre_body(i, acc):
        mb = mask_v[pl.ds(i * _L, _L)] != 0
        return acc + _popcount(mb)

    cnt_first = lax.fori_loop(0, _HALF_CHUNKS, pre_body, jnp.int32(0))
    off0 = jnp.where(c == 1, cnt_first, jnp.int32(0))

    def body(i, st):
        off, acc_mis, acc_cnt = st
        mb = mask_v[pl.ds((c * _HALF_CHUNKS + i) * _L, _L)] != 0
        m01 = jnp.where(mb, 1, 0).astype(jnp.int32)
        pv = pred_v[pl.ds(i * _L, _L)]
        g = plsc.load_expanded(tgt_v.at[pl.ds(off, _L)], mask=mb)
        spb = lax.shift_right_logical(lax.bitcast_convert_type(pv, jnp.int32), 31)
        stb = lax.shift_right_logical(lax.bitcast_convert_type(g, jnp.int32), 31)
        mis = (spb ^ stb) & m01
        return (off + _popcount(mb), acc_mis + mis, acc_cnt + m01)

    z = jnp.zeros((_L,), jnp.int32)
    _, acc_mis, acc_cnt = lax.fori_loop(0, _HALF_CHUNKS, body, (off0, z, z))

    out_v[0, :] = acc_mis.astype(jnp.float32)
    out_v[1, :] = acc_cnt.astype(jnp.float32)
    wid = s * 2 + c
    pltpu.sync_copy(out_v, out_hbm.at[wid])


def _reduce_body(p_ref, o_ref):
    mis = jnp.sum(p_ref[:, 0, :])
    cnt = jnp.sum(p_ref[:, 1, :])
    o_ref[...] = jnp.full((1, 1), 100.0 * mis / cnt, jnp.float32)


def kernel(prediction, target, mask):
    partials = _sc_partials(prediction, target, mask)
    out = pl.pallas_call(
        _reduce_body,
        out_shape=jax.ShapeDtypeStruct((1, 1), jnp.float32),
    )(partials)
    return out[0, 0]
